# Initial kernel scaffold; baseline (speedup 1.0000x reference)
import jax
import jax.numpy as jnp
from jax.experimental import pallas as pl


def kernel(x, pos_edge_index, neg_edge_index, edge_index, W0_pos, b0_pos, W0_neg, b0_neg, W1_pos, b1_pos, W1_neg, b1_neg, Wc, bc):
    raise NotImplementedError("write your pallas kernel here")


# speedup vs baseline: 1.6732x; 1.6732x over previous
import functools
import jax
import jax.numpy as jnp
from jax.experimental import pallas as pl

N = 50000
D = 128
H = 128
OUT = 2
_ROWS = 1000


def _mm_scale_body(x_ref, w_ref, ns_ref, o_ref):
    o_ref[...] = jnp.dot(x_ref[...], w_ref[...],
                         preferred_element_type=jnp.float32) * ns_ref[...]


def _mm_scale(x, W, ns):
    n = x.shape[0]
    grid = n // _ROWS
    return pl.pallas_call(
        _mm_scale_body,
        grid=(grid,),
        in_specs=[
            pl.BlockSpec((_ROWS, D), lambda i: (i, 0)),
            pl.BlockSpec((D, H), lambda i: (0, 0)),
            pl.BlockSpec((_ROWS, 1), lambda i: (i, 0)),
        ],
        out_specs=pl.BlockSpec((_ROWS, H), lambda i: (i, 0)),
        out_shape=jax.ShapeDtypeStruct((n, H), jnp.float32),
    )(x, W, ns.reshape(n, 1))


def _layer_mm_body(agg_ref, nd_ref, b_ref, w_ref, ns_ref, o_ref):
    h = jnp.maximum(agg_ref[...] * nd_ref[...] + b_ref[...], 0.0)
    o_ref[...] = jnp.dot(h, w_ref[...],
                         preferred_element_type=jnp.float32) * ns_ref[...]


def _layer_mm(agg, nd, b, W, ns):
    n = agg.shape[0]
    grid = n // _ROWS
    return pl.pallas_call(
        _layer_mm_body,
        grid=(grid,),
        in_specs=[
            pl.BlockSpec((_ROWS, H), lambda i: (i, 0)),
            pl.BlockSpec((_ROWS, 1), lambda i: (i, 0)),
            pl.BlockSpec((H,), lambda i: (0,)),
            pl.BlockSpec((H, H), lambda i: (0, 0)),
            pl.BlockSpec((_ROWS, 1), lambda i: (i, 0)),
        ],
        out_specs=pl.BlockSpec((_ROWS, H), lambda i: (i, 0)),
        out_shape=jax.ShapeDtypeStruct((n, H), jnp.float32),
    )(agg, nd.reshape(n, 1), b, W, ns.reshape(n, 1))


def _z_body(ap_ref, ndp_ref, bp_ref, an_ref, ndn_ref, bn_ref, o_ref):
    hp = jnp.maximum(ap_ref[...] * ndp_ref[...] + bp_ref[...], 0.0)
    hn = jnp.maximum(an_ref[...] * ndn_ref[...] + bn_ref[...], 0.0)
    o_ref[...] = hp - hn


def _z_combine(aggp, ndp, bp, aggn, ndn, bn):
    grid = N // _ROWS
    return pl.pallas_call(
        _z_body,
        grid=(grid,),
        in_specs=[
            pl.BlockSpec((_ROWS, H), lambda i: (i, 0)),
            pl.BlockSpec((_ROWS, 1), lambda i: (i, 0)),
            pl.BlockSpec((H,), lambda i: (0,)),
            pl.BlockSpec((_ROWS, H), lambda i: (i, 0)),
            pl.BlockSpec((_ROWS, 1), lambda i: (i, 0)),
            pl.BlockSpec((H,), lambda i: (0,)),
        ],
        out_specs=pl.BlockSpec((_ROWS, H), lambda i: (i, 0)),
        out_shape=jax.ShapeDtypeStruct((N, H), jnp.float32),
    )(aggp, ndp.reshape(N, 1), bp, aggn, ndn.reshape(N, 1), bn)


def _logits_body(zs_ref, zd_ref, wt_ref, wb_ref, bc_ref, o_ref):
    acc = jnp.dot(zs_ref[...], wt_ref[...], preferred_element_type=jnp.float32)
    acc = acc + jnp.dot(zd_ref[...], wb_ref[...],
                        preferred_element_type=jnp.float32)
    o_ref[...] = jax.nn.sigmoid(acc + bc_ref[...])


def _logits(zs, zd, Wc, bc):
    b = zs.shape[0]
    grid = b // _ROWS
    wt = Wc[:H]
    wb = Wc[H:]
    return pl.pallas_call(
        _logits_body,
        grid=(grid,),
        in_specs=[
            pl.BlockSpec((_ROWS, H), lambda i: (i, 0)),
            pl.BlockSpec((_ROWS, H), lambda i: (i, 0)),
            pl.BlockSpec((H, OUT), lambda i: (0, 0)),
            pl.BlockSpec((H, OUT), lambda i: (0, 0)),
            pl.BlockSpec((OUT,), lambda i: (0,)),
        ],
        out_specs=pl.BlockSpec((_ROWS, OUT), lambda i: (i, 0)),
        out_shape=jax.ShapeDtypeStruct((b, OUT), jnp.float32),
    )(zs, zd, wt, wb, bc)


def _norm(deg):
    return jnp.where(deg > 0, jax.lax.rsqrt(jnp.maximum(deg, 1.0)), 0.0)


def kernel(x, pos_edge_index, neg_edge_index, edge_index,
           W0_pos, b0_pos, W0_neg, b0_neg,
           W1_pos, b1_pos, W1_neg, b1_neg, Wc, bc):
    ps, pd = pos_edge_index[0], pos_edge_index[1]
    ns_, nd_ = neg_edge_index[0], neg_edge_index[1]

    ones_p = jnp.ones(ps.shape[0], dtype=jnp.float32)
    deg_ps = jax.ops.segment_sum(ones_p, ps, num_segments=N)
    deg_pd = jax.ops.segment_sum(ones_p, pd, num_segments=N)
    deg_ns = jax.ops.segment_sum(ones_p, ns_, num_segments=N)
    deg_nd = jax.ops.segment_sum(ones_p, nd_, num_segments=N)
    nrm_ps, nrm_pd = _norm(deg_ps), _norm(deg_pd)
    nrm_ns, nrm_nd = _norm(deg_ns), _norm(deg_nd)

    hs_p = _mm_scale(x, W0_pos, nrm_ps)
    hs_n = _mm_scale(x, W0_neg, nrm_ns)
    agg_p = jax.ops.segment_sum(hs_p[ps], pd, num_segments=N)
    agg_n = jax.ops.segment_sum(hs_n[ns_], nd_, num_segments=N)

    hs_p = _layer_mm(agg_p, nrm_pd, b0_pos, W1_pos, nrm_ps)
    hs_n = _layer_mm(agg_n, nrm_nd, b0_neg, W1_neg, nrm_ns)
    agg_p = jax.ops.segment_sum(hs_p[ps], pd, num_segments=N)
    agg_n = jax.ops.segment_sum(hs_n[ns_], nd_, num_segments=N)

    z = _z_combine(agg_p, nrm_pd, b1_pos, agg_n, nrm_nd, b1_neg)

    src_ids, dst_ids = edge_index[0], edge_index[1]
    probs = _logits(z[src_ids], z[dst_ids], Wc, bc)
    return (z, probs)


# speedup vs baseline: 4.8953x; 2.9258x over previous
import functools
import jax
import jax.numpy as jnp
from jax import lax
from jax.experimental import pallas as pl
from jax.experimental.pallas import tpu as pltpu
from jax.experimental.pallas import tpu_sc as plsc

N = 50000
D = 128
H = 128
OUT = 2
_ROWS = 1000

_NC = 2
_NS = 16
_NW = _NC * _NS

_B = 100000
_GC = 128
_NCHUNK = (_B + _GC - 1) // _GC
_BPAD = _NCHUNK * _GC
_TAIL = _B - (_NCHUNK - 1) * _GC


def _pair_gather_body(z_hbm, sidx_hbm, didx_hbm, outs_hbm, outd_hbm,
                      idx_v, rows_v, sem):
    wid = lax.axis_index("s") * _NC + lax.axis_index("c")
    for k in range(25):
        cid = wid + 32 * k

        @pl.when(cid < _NCHUNK)
        def _():
            base = cid * _GC
            for idx_hbm, out_hbm in ((sidx_hbm, outs_hbm),
                                     (didx_hbm, outd_hbm)):
                pltpu.sync_copy(idx_hbm.at[pl.ds(base, _GC)], idx_v)
                pltpu.async_copy(z_hbm.at[idx_v], rows_v, sem).wait()

                @pl.when(cid < _NCHUNK - 1)
                def _():
                    pltpu.sync_copy(rows_v, out_hbm.at[pl.ds(base, _GC)])

                @pl.when(cid == _NCHUNK - 1)
                def _():
                    pltpu.sync_copy(rows_v.at[pl.ds(0, _TAIL)],
                                    out_hbm.at[pl.ds(base, _TAIL)])


def _pair_gather(z, sidx, didx):
    pad = jnp.zeros((_BPAD - _B,), dtype=jnp.int32)
    sidx_p = jnp.concatenate([sidx, pad])
    didx_p = jnp.concatenate([didx, pad])
    f = pl.kernel(
        _pair_gather_body,
        out_type=(jax.ShapeDtypeStruct((_B, H), jnp.float32),
                  jax.ShapeDtypeStruct((_B, H), jnp.float32)),
        mesh=plsc.VectorSubcoreMesh(core_axis_name="c", subcore_axis_name="s"),
        compiler_params=pltpu.CompilerParams(needs_layout_passes=False),
        scratch_types=[
            pltpu.VMEM((_GC,), jnp.int32),
            pltpu.VMEM((_GC, H), jnp.float32),
            pltpu.SemaphoreType.DMA,
        ],
    )
    return f(z, sidx_p, didx_p)


E = 400000
_WPT = 12544
_EPAD = _WPT * _NW
_SEG = 6272
_SEGV = _SEG // 16
_CCAP = 13312
_NR = 4
_RNGS = (12504, 12496)
_RBASE = (0, 12504, 25000, 37504)
_RSIZE = (12504, 12496, 12504, 12496)
_ACC = 12544
_DUMP = 12536
_FL = 784
_DN = 50176


def _deg_body(i0, i1, i2, i3, deg_out, seg_a, deg, dsem):
    t = lax.axis_index("s")
    c = lax.axis_index("c")
    w = t * _NC + c
    zvec = jnp.zeros((16,), jnp.float32)
    ones = jnp.ones((16,), jnp.float32)
    for j, ref in enumerate((i0, i1, i2, i3)):
        def zb(i, _):
            deg[pl.ds(16 * i, 16)] = zvec
            return 0
        lax.fori_loop(0, _DN // 16, zb, 0)
        for s in range(2):
            pltpu.sync_copy(ref.at[pl.ds(w * _WPT + s * _SEG, _SEG)], seg_a)

            def db(i, _):
                i16 = seg_a[pl.ds(16 * i, 16)]
                m = i16 >= 0
                plsc.addupdate_scatter(deg, [i16], ones, mask=m)
                return 0
            lax.fori_loop(0, _SEGV, db, 0)
        pltpu.sync_copy(deg, deg_out.at[w, j])


def _sc_degrees(ps, pd, nsrc, ndst):
    f = pl.kernel(
        _deg_body,
        out_type=jax.ShapeDtypeStruct((_NW, 4, _DN), jnp.float32),
        mesh=plsc.VectorSubcoreMesh(core_axis_name="c", subcore_axis_name="s"),
        compiler_params=pltpu.CompilerParams(needs_layout_passes=False),
        scratch_types=[
            pltpu.VMEM((_SEG,), jnp.int32),
            pltpu.VMEM((_DN,), jnp.float32),
            pltpu.SemaphoreType.DMA,
        ],
    )
    return f(ps, pd, nsrc, ndst)


def _compact_body(srcref, dstref, cs, cl, cntout,
                  seg_a, seg_b,
                  b_s0, b_s1, b_s2, b_s3,
                  b_l0, b_l1, b_l2, b_l3,
                  cntv, dsem):
    t = lax.axis_index("s")
    c = lax.axis_index("c")
    w = t * _NC + c
    zidx = jnp.zeros((16,), jnp.int32)
    dvec = jnp.full((16,), _DUMP, jnp.int32)
    iota = lax.iota(jnp.int32, 16)
    bs = (b_s0, b_s1, b_s2, b_s3)
    bl = (b_l0, b_l1, b_l2, b_l3)

    cnts = (0,) * _NR
    for s in range(2):
        pltpu.sync_copy(srcref.at[pl.ds(w * _WPT + s * _SEG, _SEG)], seg_a)
        pltpu.sync_copy(dstref.at[pl.ds(w * _WPT + s * _SEG, _SEG)], seg_b)

        def cb(i, carry):
            s16 = seg_a[pl.ds(16 * i, 16)]
            d16 = seg_b[pl.ds(16 * i, 16)]
            out = []
            for r in range(_NR):
                l16 = d16 - _RBASE[r]
                m = (l16 >= 0) & (l16 < _RSIZE[r])
                plsc.store_compressed(bs[r].at[pl.ds(carry[r], 16)],
                                      s16, mask=m)
                plsc.store_compressed(bl[r].at[pl.ds(carry[r], 16)],
                                      l16, mask=m)
                out.append(carry[r] + jnp.sum(m.astype(jnp.int32)))
            return tuple(out)
        cnts = lax.fori_loop(0, _SEGV, cb, cnts)

    cv = jnp.zeros((16,), jnp.int32)
    for r in range(_NR):
        for k in range(9):
            bs[r][pl.ds(cnts[r] + 16 * k, 16)] = zidx
            bl[r][pl.ds(cnts[r] + 16 * k, 16)] = dvec
        pltpu.sync_copy(bs[r], cs.at[w, r])
        pltpu.sync_copy(bl[r], cl.at[w, r])
        cv = jnp.where(iota == r, cnts[r], cv)
    cntv[...] = cv
    pltpu.sync_copy(cntv, cntout.at[w])


def _sc_compact(src_p, dst_p):
    lists = jax.ShapeDtypeStruct((_NW, _NR, _CCAP), jnp.int32)
    f = pl.kernel(
        _compact_body,
        out_type=(lists, lists, jax.ShapeDtypeStruct((_NW, 16), jnp.int32)),
        mesh=plsc.VectorSubcoreMesh(core_axis_name="c", subcore_axis_name="s"),
        compiler_params=pltpu.CompilerParams(needs_layout_passes=False),
        scratch_types=[pltpu.VMEM((_SEG,), jnp.int32)] * 2
        + [pltpu.VMEM((_CCAP,), jnp.int32)] * 8
        + [pltpu.VMEM((16,), jnp.int32), pltpu.SemaphoreType.DMA],
    )
    return f(src_p, dst_p)


_GR = 128


def _scatter_body(hs_hbm, cs_hbm, cl_hbm, cnt_hbm, out_hbm,
                  gbig, lbig, gidx, sidx, rows, cntv, acc, gsem):
    t = lax.axis_index("s")
    c = lax.axis_index("c")
    iota = lax.iota(jnp.int32, 16)
    zvec = jnp.zeros((16,), jnp.float32)

    for p in range(2):
        r = c * 2 + p
        rng_size = _RNGS[p]
        out_base0 = c * 25000 + p * _RNGS[0]

        def zr(i, _):
            for l in range(8):
                rows[i, pl.ds(16 * l, 16)] = zvec
            return 0
        lax.fori_loop(0, _GR, zr, 0)
        for j in range(6):
            pltpu.sync_copy(rows, acc.at[pl.ds(t * _FL + _GR * j, _GR)])
        pltpu.sync_copy(rows.at[pl.ds(0, 16)],
                        acc.at[pl.ds(t * _FL + _GR * 6, 16)])
        plsc.subcore_barrier()

        for wl in range(2):
            w = t * 2 + wl
            pltpu.sync_copy(cnt_hbm.at[w], cntv)
            cvec = cntv[...]
            cnt = jnp.sum(jnp.where(iota == r, cvec, 0))
            nch = (cnt + _GR - 1) // _GR
            nblk = (nch + 7) // 8

            def blk(bk, _):
                pltpu.sync_copy(
                    cs_hbm.at[w, r, pl.ds(bk * 1024, 1024)], gbig)
                pltpu.sync_copy(
                    cl_hbm.at[w, r, pl.ds(bk * 1024, 1024)], lbig)
                for j in range(8):
                    q = bk * 8 + j

                    @pl.when(q < nch)
                    def _():
                        def cp(m, _):
                            gidx[pl.ds(16 * m, 16)] = (
                                gbig[pl.ds(j * _GR + 16 * m, 16)])
                            sidx[pl.ds(16 * m, 16)] = (
                                lbig[pl.ds(j * _GR + 16 * m, 16)])
                            return 0
                        lax.fori_loop(0, 8, cp, 0)
                        pltpu.async_copy(hs_hbm.at[gidx], rows,
                                         gsem).wait()
                        pltpu.sync_copy(rows, acc.at[sidx], add=True)
                return 0
            lax.fori_loop(0, nblk, blk, 0)

        plsc.subcore_barrier()
        out_base = out_base0 + t * _FL
        tail = rng_size - (_NS - 1) * _FL

        @pl.when(t < _NS - 1)
        def _():
            pltpu.sync_copy(acc.at[pl.ds(t * _FL, _FL)],
                            out_hbm.at[pl.ds(out_base, _FL)])

        @pl.when(t == _NS - 1)
        def _():
            pltpu.sync_copy(acc.at[pl.ds(t * _FL, tail)],
                            out_hbm.at[pl.ds(out_base, tail)])
        plsc.subcore_barrier()


def _sc_scatter(hs, cs, cl, cnts):
    f = pl.kernel(
        _scatter_body,
        out_type=jax.ShapeDtypeStruct((N, H), jnp.float32),
        mesh=plsc.VectorSubcoreMesh(core_axis_name="c", subcore_axis_name="s"),
        compiler_params=pltpu.CompilerParams(needs_layout_passes=False),
        scratch_types=[
            pltpu.VMEM((1024,), jnp.int32),
            pltpu.VMEM((1024,), jnp.int32),
            pltpu.VMEM((_GR,), jnp.int32),
            pltpu.VMEM((_GR,), jnp.int32),
            pltpu.VMEM((_GR, H), jnp.float32),
            pltpu.VMEM((16,), jnp.int32),
            pltpu.VMEM_SHARED((_ACC, H), jnp.float32),
            pltpu.SemaphoreType.DMA,
        ],
    )
    return f(hs, cs, cl, cnts)


def _mm_scale_body(x_ref, w_ref, ns_ref, o_ref):
    o_ref[...] = jnp.dot(x_ref[...], w_ref[...],
                         preferred_element_type=jnp.float32) * ns_ref[...]


def _mm_scale(x, W, ns):
    n = x.shape[0]
    grid = n // _ROWS
    return pl.pallas_call(
        _mm_scale_body,
        grid=(grid,),
        in_specs=[
            pl.BlockSpec((_ROWS, D), lambda i: (i, 0)),
            pl.BlockSpec((D, H), lambda i: (0, 0)),
            pl.BlockSpec((_ROWS, 1), lambda i: (i, 0)),
        ],
        out_specs=pl.BlockSpec((_ROWS, H), lambda i: (i, 0)),
        out_shape=jax.ShapeDtypeStruct((n, H), jnp.float32),
    )(x, W, ns.reshape(n, 1))


def _layer_mm_body(agg_ref, nd_ref, b_ref, w_ref, ns_ref, o_ref):
    h = jnp.maximum(agg_ref[...] * nd_ref[...] + b_ref[...], 0.0)
    o_ref[...] = jnp.dot(h, w_ref[...],
                         preferred_element_type=jnp.float32) * ns_ref[...]


def _layer_mm(agg, nd, b, W, ns):
    n = agg.shape[0]
    grid = n // _ROWS
    return pl.pallas_call(
        _layer_mm_body,
        grid=(grid,),
        in_specs=[
            pl.BlockSpec((_ROWS, H), lambda i: (i, 0)),
            pl.BlockSpec((_ROWS, 1), lambda i: (i, 0)),
            pl.BlockSpec((H,), lambda i: (0,)),
            pl.BlockSpec((H, H), lambda i: (0, 0)),
            pl.BlockSpec((_ROWS, 1), lambda i: (i, 0)),
        ],
        out_specs=pl.BlockSpec((_ROWS, H), lambda i: (i, 0)),
        out_shape=jax.ShapeDtypeStruct((n, H), jnp.float32),
    )(agg, nd.reshape(n, 1), b, W, ns.reshape(n, 1))


def _z_body(ap_ref, ndp_ref, bp_ref, an_ref, ndn_ref, bn_ref, o_ref):
    hp = jnp.maximum(ap_ref[...] * ndp_ref[...] + bp_ref[...], 0.0)
    hn = jnp.maximum(an_ref[...] * ndn_ref[...] + bn_ref[...], 0.0)
    o_ref[...] = hp - hn


def _z_combine(aggp, ndp, bp, aggn, ndn, bn):
    grid = N // _ROWS
    full = pl.BlockSpec((_ROWS, H), lambda i: (i, 0))
    vec = pl.BlockSpec((_ROWS, 1), lambda i: (i, 0))
    bias = pl.BlockSpec((H,), lambda i: (0,))
    return pl.pallas_call(
        _z_body,
        grid=(grid,),
        in_specs=[full, vec, bias, full, vec, bias],
        out_specs=pl.BlockSpec((_ROWS, H), lambda i: (i, 0)),
        out_shape=jax.ShapeDtypeStruct((N, H), jnp.float32),
    )(aggp, ndp.reshape(N, 1), bp, aggn, ndn.reshape(N, 1), bn)


def _logits_body(zs_ref, zd_ref, wt_ref, wb_ref, bc_ref, o_ref):
    acc = jnp.dot(zs_ref[...], wt_ref[...], preferred_element_type=jnp.float32)
    acc = acc + jnp.dot(zd_ref[...], wb_ref[...],
                        preferred_element_type=jnp.float32)
    o_ref[...] = jax.nn.sigmoid(acc + bc_ref[...])


def _logits(zs, zd, Wc, bc):
    b = zs.shape[0]
    grid = b // _ROWS
    wt = Wc[:H]
    wb = Wc[H:]
    return pl.pallas_call(
        _logits_body,
        grid=(grid,),
        in_specs=[
            pl.BlockSpec((_ROWS, H), lambda i: (i, 0)),
            pl.BlockSpec((_ROWS, H), lambda i: (i, 0)),
            pl.BlockSpec((H, OUT), lambda i: (0, 0)),
            pl.BlockSpec((H, OUT), lambda i: (0, 0)),
            pl.BlockSpec((OUT,), lambda i: (0,)),
        ],
        out_specs=pl.BlockSpec((_ROWS, OUT), lambda i: (i, 0)),
        out_shape=jax.ShapeDtypeStruct((b, OUT), jnp.float32),
    )(zs, zd, wt, wb, bc)


def _norm(deg):
    return jnp.where(deg > 0, jax.lax.rsqrt(jnp.maximum(deg, 1.0)), 0.0)


def kernel(x, pos_edge_index, neg_edge_index, edge_index,
           W0_pos, b0_pos, W0_neg, b0_neg,
           W1_pos, b1_pos, W1_neg, b1_neg, Wc, bc):
    epad = jnp.full((_EPAD - E,), -1, dtype=jnp.int32)
    ps = jnp.concatenate([pos_edge_index[0], epad])
    pd = jnp.concatenate([pos_edge_index[1], epad])
    ns_ = jnp.concatenate([neg_edge_index[0], epad])
    nd_ = jnp.concatenate([neg_edge_index[1], epad])

    degs = _sc_degrees(ps, pd, ns_, nd_).sum(axis=0)[:, :N]
    cs_p, cl_p, cnt_p = _sc_compact(ps, pd)
    cs_n, cl_n, cnt_n = _sc_compact(ns_, nd_)
    nrm_ps, nrm_pd = _norm(degs[0]), _norm(degs[1])
    nrm_ns, nrm_nd = _norm(degs[2]), _norm(degs[3])

    hs_p = _mm_scale(x, W0_pos, nrm_ps)
    hs_n = _mm_scale(x, W0_neg, nrm_ns)
    agg_p = _sc_scatter(hs_p, cs_p, cl_p, cnt_p)
    agg_n = _sc_scatter(hs_n, cs_n, cl_n, cnt_n)

    hs_p = _layer_mm(agg_p, nrm_pd, b0_pos, W1_pos, nrm_ps)
    hs_n = _layer_mm(agg_n, nrm_nd, b0_neg, W1_neg, nrm_ns)
    agg_p = _sc_scatter(hs_p, cs_p, cl_p, cnt_p)
    agg_n = _sc_scatter(hs_n, cs_n, cl_n, cnt_n)

    z = _z_combine(agg_p, nrm_pd, b1_pos, agg_n, nrm_nd, b1_neg)

    src_ids, dst_ids = edge_index[0], edge_index[1]
    zs, zd = _pair_gather(z, src_ids, dst_ids)
    probs = _logits(zs, zd, Wc, bc)
    return (z, probs)
